# baseline (speedup 1.0000x reference)
import functools

import jax
import jax.numpy as jnp
from jax import lax
from jax.experimental import pallas as pl
from jax.experimental.pallas import tpu as pltpu
from jax.experimental.pallas import tpu_sc as plsc

BATCH = 16384
D = 32
N_NEGS = 20
NC = 2
NS = 16
NW = NC * NS
BPW = BATCH // NW
CB = 16
NCHUNK = BPW // CB
ROWS = CB * N_NEGS
GPC = ROWS // 16
L = 16


def _neg_block(c, g5, hbuf, t_buf, negbuf, part_buf):
    base_b = c * CB + g5 * 4
    for r in range(5):
        lanes = [(r * L + l) // N_NEGS for l in range(L)]
        b0, b1 = lanes[0], lanes[-1]
        n0 = lanes.count(b0)
        mask = lax.iota(jnp.int32, L) < n0
        h0 = [hbuf[base_b + b0, pl.ds(pl.multiple_of(k * L, L), L)]
              for k in range(D // L)]
        if b1 != b0:
            h1 = [hbuf[base_b + b1, pl.ds(pl.multiple_of(k * L, L), L)]
                  for k in range(D // L)]
        flat = (jnp.full((L,), (g5 * 5 + r) * L, jnp.int32)
                + lax.iota(jnp.int32, L))
        off = pl.multiple_of((g5 * 5 + r) * L, L)
        if part_buf is None:
            acc = jnp.zeros((L,), jnp.float32)
        else:
            acc = part_buf[pl.ds(off, L)]
        for k in range(D // 2):
            kc = jnp.full((L,), k, jnp.int32)
            pv = plsc.load_gather(t_buf, [flat, kc])
            ev, od = plsc.unpack(plsc.bitcast(pv, jnp.bfloat16),
                                 format=plsc.PackFormat.INTERLEAVED,
                                 preferred_element_type=jnp.float32)
            half = (2 * k) // L
            le = jnp.full((L,), (2 * k) % L, jnp.int32)
            lo = jnp.full((L,), (2 * k) % L + 1, jnp.int32)
            hve = jnp.take_along_axis(h0[half], le, axis=0)
            hvo = jnp.take_along_axis(h0[half], lo, axis=0)
            if b1 != b0:
                hve = jnp.where(mask, hve,
                                jnp.take_along_axis(h1[half], le, axis=0))
                hvo = jnp.where(mask, hvo,
                                jnp.take_along_axis(h1[half], lo, axis=0))
            acc = acc + hve * ev + hvo * od
        negbuf[pl.ds(off, L)] = acc


def _make_body(with_partial):
    def body(*refs):
        if with_partial:
            (hrows_r, items_r, negs_r, t_r, ppos_r, pneg_r,
             outpos_r, outneg_r,
             idx_t, idx_n, hbuf, tbuf,
             nA, nB, partA, partB, pposbuf, posbuf, negbuf,
             semP, semA, semB) = refs
        else:
            (hrows_r, items_r, negs_r, t_r,
             outpos_r, outneg_r,
             idx_t, idx_n, hbuf, tbuf,
             nA, nB, pposbuf, posbuf, negbuf,
             semP, semA, semB) = refs
            ppos_r = pneg_r = partA = partB = None

        wid = lax.axis_index("s") * NC + lax.axis_index("c")
        nbase = wid * (BPW * N_NEGS)

        pltpu.sync_copy(items_r.at[pl.ds(wid * BPW, BPW)], idx_t)
        pltpu.sync_copy(negs_r.at[pl.ds(nbase, BPW * N_NEGS)], idx_n)

        pltpu.async_copy(hrows_r.at[pl.ds(wid * BPW, BPW)], hbuf, semP)
        for k in range(BPW // 128):
            sl = pl.ds(k * 128, 128)
            pltpu.async_copy(t_r.at[idx_t.at[sl]], tbuf.at[sl], semP)
        if with_partial:
            pltpu.async_copy(ppos_r.at[pl.ds(wid * BPW, BPW)], pposbuf, semP)

        def fire(c, n_buf, part, sem):
            base = c * ROWS
            for off, n in ((0, 128), (128, 128), (256, 64)):
                row = pl.ds(base + off, n)
                pltpu.async_copy(t_r.at[idx_n.at[row]],
                                 n_buf.at[pl.ds(off, n)], sem)
            if with_partial:
                pltpu.async_copy(
                    pneg_r.at[pl.ds(pl.multiple_of(nbase + base, 8), ROWS)],
                    part, sem)

        def drain(c, n_buf, part, sem):
            pltpu.make_async_copy(t_r.at[pl.ds(0, ROWS)], n_buf, sem).wait()
            if with_partial:
                pltpu.make_async_copy(
                    pneg_r.at[pl.ds(0, ROWS)], part, sem).wait()

        fire(0, nA, partA, semA)

        pltpu.make_async_copy(hrows_r.at[pl.ds(0, BPW)], hbuf, semP).wait()
        pltpu.make_async_copy(t_r.at[pl.ds(0, BPW)], tbuf, semP).wait()
        if with_partial:
            pltpu.make_async_copy(
                ppos_r.at[pl.ds(0, BPW)], pposbuf, semP).wait()

        def pos_group(g, carry):
            flat = (jnp.full((L,), g * L, jnp.int32)
                    + lax.iota(jnp.int32, L))
            off = pl.multiple_of(g * L, L)
            if with_partial:
                acc = pposbuf[pl.ds(off, L)]
            else:
                acc = jnp.zeros((L,), jnp.float32)
            for k in range(D // 2):
                kc = jnp.full((L,), k, jnp.int32)
                tv = plsc.load_gather(tbuf, [flat, kc])
                ev, od = plsc.unpack(plsc.bitcast(tv, jnp.bfloat16),
                                     format=plsc.PackFormat.INTERLEAVED,
                                     preferred_element_type=jnp.float32)
                hve = plsc.load_gather(
                    hbuf, [flat, jnp.full((L,), 2 * k, jnp.int32)])
                hvo = plsc.load_gather(
                    hbuf, [flat, jnp.full((L,), 2 * k + 1, jnp.int32)])
                acc = acc + hve * ev + hvo * od
            posbuf[pl.ds(off, L)] = acc
            return carry

        lax.fori_loop(0, BPW // L, pos_group, 0)
        pltpu.sync_copy(posbuf, outpos_r.at[pl.ds(wid * BPW, BPW)])

        def compute_chunk(c, n_buf, part):
            def blk(g5, carry):
                _neg_block(c, g5, hbuf, n_buf, negbuf, part)
                return carry

            lax.fori_loop(0, GPC // 5, blk, 0)
            base = pl.multiple_of(nbase + c * ROWS, 8)
            pltpu.sync_copy(negbuf, outneg_r.at[pl.ds(base, ROWS)])

        def pair(t, carry):
            c0 = t * 2
            fire(c0 + 1, nB, partB, semB)
            drain(c0, nA, partA, semA)
            compute_chunk(c0, nA, partA)

            @pl.when(t + 1 < NCHUNK // 2)
            def _():
                fire(c0 + 2, nA, partA, semA)

            drain(c0 + 1, nB, partB, semB)
            compute_chunk(c0 + 1, nB, partB)
            return carry

        lax.fori_loop(0, NCHUNK // 2, pair, 0)

    return body


def _scratch(with_partial):
    s = [
        pltpu.VMEM((BPW,), jnp.int32),
        pltpu.VMEM((BPW * N_NEGS,), jnp.int32),
        pltpu.VMEM((BPW, D), jnp.float32),
        pltpu.VMEM((BPW, D // 2), jnp.int32),
        pltpu.VMEM((ROWS, D // 2), jnp.int32),
        pltpu.VMEM((ROWS, D // 2), jnp.int32),
    ]
    if with_partial:
        s += [
            pltpu.VMEM((ROWS,), jnp.float32),
            pltpu.VMEM((ROWS,), jnp.float32),
        ]
    s += [
        pltpu.VMEM((BPW,), jnp.float32),
        pltpu.VMEM((BPW,), jnp.float32),
        pltpu.VMEM((ROWS,), jnp.float32),
        pltpu.SemaphoreType.DMA,
        pltpu.SemaphoreType.DMA,
        pltpu.SemaphoreType.DMA,
    ]
    return s


_OUT = [
    jax.ShapeDtypeStruct((BATCH,), jnp.float32),
    jax.ShapeDtypeStruct((BATCH * N_NEGS,), jnp.float32),
]
_PARAMS = pltpu.CompilerParams(needs_layout_passes=False,
                               use_tc_tiling_on_sc=False)


@jax.jit
def _run(hrows, itemsi, itemsj, negs, P, Q):
    mesh = plsc.VectorSubcoreMesh(core_axis_name="c", subcore_axis_name="s",
                                  num_cores=NC, num_subcores=NS)
    fp = pl.kernel(_make_body(False), out_type=_OUT, mesh=mesh,
                   compiler_params=_PARAMS, scratch_types=_scratch(False))
    fq = pl.kernel(_make_body(True), out_type=_OUT, mesh=mesh,
                   compiler_params=_PARAMS, scratch_types=_scratch(True))
    Pi = lax.bitcast_convert_type(
        P.astype(jnp.bfloat16).reshape(-1, D // 2, 2), jnp.int32)
    Qi = lax.bitcast_convert_type(
        Q.astype(jnp.bfloat16).reshape(-1, D // 2, 2), jnp.int32)
    ppos, pneg = fp(hrows, itemsi, negs, Pi)
    return fq(hrows, itemsj, negs, Qi, ppos, pneg)


def kernel(users, items_i, items_j, negs, H, P, Q):
    h_u = jnp.take(H, users, axis=0)
    pos, neg = _run(h_u, items_i, items_j, negs.reshape(-1), P, Q)
    return pos, neg.reshape(BATCH, N_NEGS)


# speedup vs baseline: 1.3895x; 1.3895x over previous
import jax
import jax.numpy as jnp
from jax import lax
from jax.experimental import pallas as pl
from jax.experimental.pallas import tpu as pltpu
from jax.experimental.pallas import tpu_sc as plsc

BATCH = 16384
D = 32
N_NEGS = 20
NC = 2
NS = 16
NW = NC * NS
BPW = BATCH // NW
CB = 16
NCHUNK = BPW // CB
ROWS = CB * N_NEGS
GPC = ROWS // 16
L = 16
HD = D // 2


def _convert_rows(src_bf16, dst_f32, nrows):
    def row(rr, carry):
        x = src_bf16[rr, :]
        ev, od = plsc.unpack(x, format=plsc.PackFormat.INTERLEAVED,
                             preferred_element_type=jnp.float32)
        dst_f32[rr, pl.ds(0, L)] = ev
        dst_f32[rr, pl.ds(L, L)] = od
        return carry

    lax.fori_loop(0, nrows, row, 0)


def _neg_block(c, g5, hbuf, tf_buf, negbuf, part_buf):
    base_b = c * CB + g5 * 4
    for r in range(5):
        lanes = [(r * L + l) // N_NEGS for l in range(L)]
        b0, b1 = lanes[0], lanes[-1]
        n0 = lanes.count(b0)
        mask = lax.iota(jnp.int32, L) < n0
        h0 = [hbuf[base_b + b0, pl.ds(pl.multiple_of(k * L, L), L)]
              for k in range(D // L)]
        if b1 != b0:
            h1 = [hbuf[base_b + b1, pl.ds(pl.multiple_of(k * L, L), L)]
                  for k in range(D // L)]
        flat = (jnp.full((L,), (g5 * 5 + r) * L, jnp.int32)
                + lax.iota(jnp.int32, L))
        off = pl.multiple_of((g5 * 5 + r) * L, L)
        if part_buf is None:
            acc = jnp.zeros((L,), jnp.float32)
        else:
            acc = part_buf[pl.ds(off, L)]
        for k in range(HD):
            ev = plsc.load_gather(tf_buf, [flat, jnp.full((L,), k, jnp.int32)])
            od = plsc.load_gather(tf_buf,
                                  [flat, jnp.full((L,), L + k, jnp.int32)])
            half = (2 * k) // L
            le = jnp.full((L,), (2 * k) % L, jnp.int32)
            lo = jnp.full((L,), (2 * k) % L + 1, jnp.int32)
            hve = jnp.take_along_axis(h0[half], le, axis=0)
            hvo = jnp.take_along_axis(h0[half], lo, axis=0)
            if b1 != b0:
                hve = jnp.where(mask, hve,
                                jnp.take_along_axis(h1[half], le, axis=0))
                hvo = jnp.where(mask, hvo,
                                jnp.take_along_axis(h1[half], lo, axis=0))
            acc = acc + hve * ev + hvo * od
        negbuf[pl.ds(off, L)] = acc


def _make_body(with_partial):
    def body(*refs):
        if with_partial:
            (hrows_r, items_r, negs_r, t_r, ppos_r, pneg_r,
             outpos_r, outneg_r,
             idx_t, idx_n, hbuf, tbuf, tposf,
             nA, nB, nfbuf, partA, partB, pposbuf, posbuf, negbuf,
             semP, semA, semB) = refs
        else:
            (hrows_r, items_r, negs_r, t_r,
             outpos_r, outneg_r,
             idx_t, idx_n, hbuf, tbuf, tposf,
             nA, nB, nfbuf, pposbuf, posbuf, negbuf,
             semP, semA, semB) = refs
            ppos_r = pneg_r = partA = partB = None

        wid = lax.axis_index("s") * NC + lax.axis_index("c")
        nbase = wid * (BPW * N_NEGS)

        pltpu.sync_copy(items_r.at[pl.ds(wid * BPW, BPW)], idx_t)
        pltpu.sync_copy(negs_r.at[pl.ds(nbase, BPW * N_NEGS)], idx_n)

        pltpu.async_copy(hrows_r.at[pl.ds(wid * BPW, BPW)], hbuf, semP)
        for k in range(BPW // 128):
            sl = pl.ds(k * 128, 128)
            pltpu.async_copy(t_r.at[idx_t.at[sl]], tbuf.at[sl], semP)
        if with_partial:
            pltpu.async_copy(ppos_r.at[pl.ds(wid * BPW, BPW)], pposbuf, semP)

        def fire(c, n_buf, part, sem):
            base = c * ROWS
            for off, n in ((0, 128), (128, 128), (256, 64)):
                row = pl.ds(base + off, n)
                pltpu.async_copy(t_r.at[idx_n.at[row]],
                                 n_buf.at[pl.ds(off, n)], sem)
            if with_partial:
                pltpu.async_copy(
                    pneg_r.at[pl.ds(pl.multiple_of(nbase + base, 8), ROWS)],
                    part, sem)

        def drain(n_buf, part, sem):
            pltpu.make_async_copy(t_r.at[pl.ds(0, ROWS)], n_buf, sem).wait()
            if with_partial:
                pltpu.make_async_copy(
                    pneg_r.at[pl.ds(0, ROWS)], part, sem).wait()

        fire(0, nA, partA, semA)

        pltpu.make_async_copy(hrows_r.at[pl.ds(0, BPW)], hbuf, semP).wait()
        pltpu.make_async_copy(t_r.at[pl.ds(0, BPW)], tbuf, semP).wait()
        if with_partial:
            pltpu.make_async_copy(
                ppos_r.at[pl.ds(0, BPW)], pposbuf, semP).wait()

        _convert_rows(tbuf, tposf, BPW)

        def pos_group(g, carry):
            flat = (jnp.full((L,), g * L, jnp.int32)
                    + lax.iota(jnp.int32, L))
            off = pl.multiple_of(g * L, L)
            if with_partial:
                acc = pposbuf[pl.ds(off, L)]
            else:
                acc = jnp.zeros((L,), jnp.float32)
            for k in range(HD):
                ev = plsc.load_gather(
                    tposf, [flat, jnp.full((L,), k, jnp.int32)])
                od = plsc.load_gather(
                    tposf, [flat, jnp.full((L,), L + k, jnp.int32)])
                hve = plsc.load_gather(
                    hbuf, [flat, jnp.full((L,), 2 * k, jnp.int32)])
                hvo = plsc.load_gather(
                    hbuf, [flat, jnp.full((L,), 2 * k + 1, jnp.int32)])
                acc = acc + hve * ev + hvo * od
            posbuf[pl.ds(off, L)] = acc
            return carry

        lax.fori_loop(0, BPW // L, pos_group, 0)
        pltpu.sync_copy(posbuf, outpos_r.at[pl.ds(wid * BPW, BPW)])

        def compute_chunk(c, n_buf, part):
            _convert_rows(n_buf, nfbuf, ROWS)

            def blk(g5, carry):
                _neg_block(c, g5, hbuf, nfbuf, negbuf, part)
                return carry

            lax.fori_loop(0, GPC // 5, blk, 0)
            base = pl.multiple_of(nbase + c * ROWS, 8)
            pltpu.sync_copy(negbuf, outneg_r.at[pl.ds(base, ROWS)])

        def pair(t, carry):
            c0 = t * 2
            fire(c0 + 1, nB, partB, semB)
            drain(nA, partA, semA)
            compute_chunk(c0, nA, partA)

            @pl.when(t + 1 < NCHUNK // 2)
            def _():
                fire(c0 + 2, nA, partA, semA)

            drain(nB, partB, semB)
            compute_chunk(c0 + 1, nB, partB)
            return carry

        lax.fori_loop(0, NCHUNK // 2, pair, 0)

    return body


def _scratch(with_partial):
    s = [
        pltpu.VMEM((BPW,), jnp.int32),
        pltpu.VMEM((BPW * N_NEGS,), jnp.int32),
        pltpu.VMEM((BPW, D), jnp.float32),
        pltpu.VMEM((BPW, D), jnp.bfloat16),
        pltpu.VMEM((BPW, D), jnp.float32),
        pltpu.VMEM((ROWS, D), jnp.bfloat16),
        pltpu.VMEM((ROWS, D), jnp.bfloat16),
        pltpu.VMEM((ROWS, D), jnp.float32),
    ]
    if with_partial:
        s += [
            pltpu.VMEM((ROWS,), jnp.float32),
            pltpu.VMEM((ROWS,), jnp.float32),
        ]
    s += [
        pltpu.VMEM((BPW,), jnp.float32),
        pltpu.VMEM((BPW,), jnp.float32),
        pltpu.VMEM((ROWS,), jnp.float32),
        pltpu.SemaphoreType.DMA,
        pltpu.SemaphoreType.DMA,
        pltpu.SemaphoreType.DMA,
    ]
    return s


_OUT = [
    jax.ShapeDtypeStruct((BATCH,), jnp.float32),
    jax.ShapeDtypeStruct((BATCH * N_NEGS,), jnp.float32),
]
_PARAMS = pltpu.CompilerParams(needs_layout_passes=False,
                               use_tc_tiling_on_sc=False)


@jax.jit
def _run(hrows, itemsi, itemsj, negs, P, Q):
    mesh = plsc.VectorSubcoreMesh(core_axis_name="c", subcore_axis_name="s",
                                  num_cores=NC, num_subcores=NS)
    fp = pl.kernel(_make_body(False), out_type=_OUT, mesh=mesh,
                   compiler_params=_PARAMS, scratch_types=_scratch(False))
    fq = pl.kernel(_make_body(True), out_type=_OUT, mesh=mesh,
                   compiler_params=_PARAMS, scratch_types=_scratch(True))
    ppos, pneg = fp(hrows, itemsi, negs, P.astype(jnp.bfloat16))
    return fq(hrows, itemsj, negs, Q.astype(jnp.bfloat16), ppos, pneg)


def kernel(users, items_i, items_j, negs, H, P, Q):
    h_u = jnp.take(H, users, axis=0)
    pos, neg = _run(h_u, items_i, items_j, negs.reshape(-1), P, Q)
    return pos, neg.reshape(BATCH, N_NEGS)


# speedup vs baseline: 1.8963x; 1.3648x over previous
import functools

import jax
import jax.numpy as jnp
from jax import lax
from jax.experimental import pallas as pl
from jax.experimental.pallas import tpu as pltpu
from jax.experimental.pallas import tpu_sc as plsc

BATCH = 16384
D = 32
N_NEGS = 20
NC = 2
NS = 16
NW = NC * NS
BPW = BATCH // NW
CB = 32
NCHUNK = BPW // CB
ROWS = CB * N_NEGS
GPC = ROWS // 16
L = 16


def _neg_block(c, g5, hbuf, t_buf, negbuf, part_buf):
    base_b = c * CB + g5 * 4
    for r in range(5):
        lanes = [(r * L + l) // N_NEGS for l in range(L)]
        b0, b1 = lanes[0], lanes[-1]
        n0 = lanes.count(b0)
        mask = lax.iota(jnp.int32, L) < n0
        h0 = [hbuf[base_b + b0, pl.ds(pl.multiple_of(k * L, L), L)]
              for k in range(D // L)]
        if b1 != b0:
            h1 = [hbuf[base_b + b1, pl.ds(pl.multiple_of(k * L, L), L)]
                  for k in range(D // L)]
        flat = (jnp.full((L,), (g5 * 5 + r) * L, jnp.int32)
                + lax.iota(jnp.int32, L))
        off = pl.multiple_of((g5 * 5 + r) * L, L)
        if part_buf is None:
            acc = jnp.zeros((L,), jnp.float32)
        else:
            acc = part_buf[pl.ds(off, L)]
        for d in range(D):
            dc = jnp.full((L,), d, jnp.int32)
            lane = jnp.full((L,), d % L, jnp.int32)
            hv = jnp.take_along_axis(h0[d // L], lane, axis=0)
            if b1 != b0:
                hv1 = jnp.take_along_axis(h1[d // L], lane, axis=0)
                hv = jnp.where(mask, hv, hv1)
            av = plsc.load_gather(t_buf, [flat, dc])
            acc = acc + hv * av
        negbuf[pl.ds(off, L)] = acc


def _make_body(with_partial):
    def body(*refs):
        if with_partial:
            (hrows_r, items_r, negs_r, t_r, ppos_r, pneg_r,
             outpos_r, outneg_r,
             idx_t, idx_n, hbuf, tbuf,
             nA, nB, partA, partB, pposbuf, posbuf, negbuf,
             semP, semA, semB) = refs
        else:
            (hrows_r, items_r, negs_r, t_r,
             outpos_r, outneg_r,
             idx_t, idx_n, hbuf, tbuf,
             nA, nB, pposbuf, posbuf, negbuf,
             semP, semA, semB) = refs
            ppos_r = pneg_r = partA = partB = None

        wid = lax.axis_index("s") * NC + lax.axis_index("c")
        nbase = wid * (BPW * N_NEGS)

        pltpu.sync_copy(items_r.at[pl.ds(wid * BPW, BPW)], idx_t)
        pltpu.sync_copy(negs_r.at[pl.ds(nbase, BPW * N_NEGS)], idx_n)

        pltpu.async_copy(hrows_r.at[pl.ds(wid * BPW, BPW)], hbuf, semP)
        for k in range(BPW // 128):
            sl = pl.ds(k * 128, 128)
            pltpu.async_copy(t_r.at[idx_t.at[sl]], tbuf.at[sl], semP)
        if with_partial:
            pltpu.async_copy(ppos_r.at[pl.ds(wid * BPW, BPW)], pposbuf, semP)

        def fire(c, n_buf, part, sem):
            base = c * ROWS
            for off, n in ((0, 128), (128, 128), (256, 128),
                           (384, 128), (512, 128)):
                row = pl.ds(base + off, n)
                pltpu.async_copy(t_r.at[idx_n.at[row]],
                                 n_buf.at[pl.ds(off, n)], sem)
            if with_partial:
                pltpu.async_copy(
                    pneg_r.at[pl.ds(pl.multiple_of(nbase + base, 8), ROWS)],
                    part, sem)

        def drain(c, n_buf, part, sem):
            pltpu.make_async_copy(t_r.at[pl.ds(0, ROWS)], n_buf, sem).wait()
            if with_partial:
                pltpu.make_async_copy(
                    pneg_r.at[pl.ds(0, ROWS)], part, sem).wait()

        fire(0, nA, partA, semA)

        pltpu.make_async_copy(hrows_r.at[pl.ds(0, BPW)], hbuf, semP).wait()
        pltpu.make_async_copy(t_r.at[pl.ds(0, BPW)], tbuf, semP).wait()
        if with_partial:
            pltpu.make_async_copy(
                ppos_r.at[pl.ds(0, BPW)], pposbuf, semP).wait()

        def pos_group(g, carry):
            flat = (jnp.full((L,), g * L, jnp.int32)
                    + lax.iota(jnp.int32, L))
            off = pl.multiple_of(g * L, L)
            if with_partial:
                acc = pposbuf[pl.ds(off, L)]
            else:
                acc = jnp.zeros((L,), jnp.float32)
            for d in range(D):
                dc = jnp.full((L,), d, jnp.int32)
                hv = plsc.load_gather(hbuf, [flat, dc])
                tv = plsc.load_gather(tbuf, [flat, dc])
                acc = acc + hv * tv
            posbuf[pl.ds(off, L)] = acc
            return carry

        lax.fori_loop(0, BPW // L, pos_group, 0)
        pltpu.sync_copy(posbuf, outpos_r.at[pl.ds(wid * BPW, BPW)])

        def compute_chunk(c, n_buf, part):
            def blk(g5, carry):
                _neg_block(c, g5, hbuf, n_buf, negbuf, part)
                return carry

            lax.fori_loop(0, GPC // 5, blk, 0)
            base = pl.multiple_of(nbase + c * ROWS, 8)
            pltpu.sync_copy(negbuf, outneg_r.at[pl.ds(base, ROWS)])

        def pair(t, carry):
            c0 = t * 2
            fire(c0 + 1, nB, partB, semB)
            drain(c0, nA, partA, semA)
            compute_chunk(c0, nA, partA)

            @pl.when(t + 1 < NCHUNK // 2)
            def _():
                fire(c0 + 2, nA, partA, semA)

            drain(c0 + 1, nB, partB, semB)
            compute_chunk(c0 + 1, nB, partB)
            return carry

        lax.fori_loop(0, NCHUNK // 2, pair, 0)

    return body


def _scratch(with_partial):
    s = [
        pltpu.VMEM((BPW,), jnp.int32),
        pltpu.VMEM((BPW * N_NEGS,), jnp.int32),
        pltpu.VMEM((BPW, D), jnp.float32),
        pltpu.VMEM((BPW, D), jnp.float32),
        pltpu.VMEM((ROWS, D), jnp.float32),
        pltpu.VMEM((ROWS, D), jnp.float32),
    ]
    if with_partial:
        s += [
            pltpu.VMEM((ROWS,), jnp.float32),
            pltpu.VMEM((ROWS,), jnp.float32),
        ]
    s += [
        pltpu.VMEM((BPW,), jnp.float32),
        pltpu.VMEM((BPW,), jnp.float32),
        pltpu.VMEM((ROWS,), jnp.float32),
        pltpu.SemaphoreType.DMA,
        pltpu.SemaphoreType.DMA,
        pltpu.SemaphoreType.DMA,
    ]
    return s


_OUT = [
    jax.ShapeDtypeStruct((BATCH,), jnp.float32),
    jax.ShapeDtypeStruct((BATCH * N_NEGS,), jnp.float32),
]
_PARAMS = pltpu.CompilerParams(needs_layout_passes=False,
                               use_tc_tiling_on_sc=False)


@jax.jit
def _run(hrows, itemsi, itemsj, negs, P, Q):
    mesh = plsc.VectorSubcoreMesh(core_axis_name="c", subcore_axis_name="s",
                                  num_cores=NC, num_subcores=NS)
    fp = pl.kernel(_make_body(False), out_type=_OUT, mesh=mesh,
                   compiler_params=_PARAMS, scratch_types=_scratch(False))
    fq = pl.kernel(_make_body(True), out_type=_OUT, mesh=mesh,
                   compiler_params=_PARAMS, scratch_types=_scratch(True))
    ppos, pneg = fp(hrows, itemsi, negs, P)
    return fq(hrows, itemsj, negs, Q, ppos, pneg)


def kernel(users, items_i, items_j, negs, H, P, Q):
    h_u = jnp.take(H, users, axis=0)
    pos, neg = _run(h_u, items_i, items_j, negs.reshape(-1), P, Q)
    return pos, neg.reshape(BATCH, N_NEGS)
